# baseline (speedup 1.0000x reference)
import jax
import jax.numpy as jnp
from jax import lax
from jax.experimental import pallas as pl
from jax.experimental.pallas import tpu as pltpu, tpu_sc as plsc

_V = 30522
_D = 768
_H = 256
_B = 256
_E = 3
_OUT = 3
_PAD = 128
_NW = 32
_GG = _B // _NW
_GE = _E * _B // _NW


def _sc_gather_body(emb_g_hbm, emb_e_hbm, idx_hbm,
                    out_g_hbm, out_e_hbm,
                    idx_v, rowsg_v, rowse_v, semg, seme):
    wid = lax.axis_index("s") * 2 + lax.axis_index("c")
    pltpu.sync_copy(idx_hbm.at[pl.ds(wid * _GG, _GG)], idx_v)
    cp_g = pltpu.async_copy(emb_g_hbm.at[idx_v], rowsg_v, semg)
    cps = [pltpu.async_copy(emb_e_hbm.at[i].at[idx_v], rowse_v.at[i], seme)
           for i in range(_E)]
    cp_g.wait()
    pltpu.sync_copy(rowsg_v, out_g_hbm.at[pl.ds(wid * _GG, _GG)])
    for i in range(_E):
        cps[i].wait()
        pltpu.sync_copy(rowse_v.at[i], out_e_hbm.at[i, pl.ds(wid * _GG, _GG)])


def _make_sc_gather():
    return pl.kernel(
        _sc_gather_body,
        out_type=(jax.ShapeDtypeStruct((_B, _D), jnp.float32),
                  jax.ShapeDtypeStruct((_E, _B, _D), jnp.float32)),
        mesh=plsc.VectorSubcoreMesh(core_axis_name="c", subcore_axis_name="s"),
        scratch_types=[
            pltpu.VMEM((_GG,), jnp.int32),
            pltpu.VMEM((_GG, _D), jnp.float32),
            pltpu.VMEM((_E, _GG, _D), jnp.float32),
            pltpu.SemaphoreType.DMA,
            pltpu.SemaphoreType.DMA,
        ],
    )


def _ln_rows(x, gamma, beta):
    mu = jnp.mean(x, axis=-1, keepdims=True)
    v = jnp.mean((x - mu) ** 2, axis=-1, keepdims=True)
    return (x - mu) * lax.rsqrt(v + 1e-5) * gamma + beta


def _tc_body(xg_ref, xe_ref, m0_ref, Wg_ref, bg_ref, We_ref, be_ref,
             Wg1_ref, bg1_ref, gga_ref, gbe_ref, Wg2_ref, bg2_ref,
             Wc1_ref, bc1_ref, cga_ref, cbe_ref, Wc2_ref, bc2_ref,
             out_c_ref, out_p_ref):
    f32 = jnp.float32
    m0 = m0_ref[...]
    xg = xg_ref[...] * m0
    h = jax.nn.gelu(jnp.dot(xg, Wg_ref[...], preferred_element_type=f32)
                    + bg_ref[...].reshape(1, _D))
    g = (jnp.dot(h, Wg1_ref[...], preferred_element_type=f32)
         + bg1_ref[...].reshape(1, _H))
    g = jax.nn.gelu(_ln_rows(g, gga_ref[...].reshape(1, _H),
                             gbe_ref[...].reshape(1, _H)))
    logits = (jnp.dot(g, Wg2_ref[...], preferred_element_type=f32)
              + bg2_ref[...].reshape(1, _OUT))
    lmax = jnp.max(logits, axis=-1, keepdims=True)
    e = jnp.exp(logits - lmax)
    p = e / jnp.sum(e, axis=-1, keepdims=True)
    out_p_ref[...] = p
    p0, p1, p2 = p[:, 0:1], p[:, 1:2], p[:, 2:3]
    excl2 = (p2 <= p0) & (p2 <= p1)
    excl1 = jnp.logical_not(excl2) & (p1 <= p0) & (p1 < p2)
    excl0 = jnp.logical_not(excl2) & jnp.logical_not(excl1)
    w0 = jnp.where(excl0, 0.0, p0)
    w1 = jnp.where(excl1, 0.0, p1)
    w2 = jnp.where(excl2, 0.0, p2)
    denom = w0 + w1 + w2
    ws = (w0 / denom, w1 / denom, w2 / denom)
    acc = jnp.zeros((_B, _D), f32)
    for i in range(_E):
        xe = xe_ref[i] * m0
        he = jax.nn.gelu(jnp.dot(xe, We_ref[i], preferred_element_type=f32)
                         + be_ref[i].reshape(1, _D))
        acc = acc + he * ws[i]
    c = (jnp.dot(acc, Wc1_ref[...], preferred_element_type=f32)
         + bc1_ref[...].reshape(1, _H))
    c = jax.nn.gelu(_ln_rows(c, cga_ref[...].reshape(1, _H),
                             cbe_ref[...].reshape(1, _H)))
    out_c_ref[...] = (jnp.dot(c, Wc2_ref[...], preferred_element_type=f32)
                      + bc2_ref[...].reshape(1, _OUT))


_tc_dense = pl.pallas_call(
    _tc_body,
    out_shape=(jax.ShapeDtypeStruct((_B, _OUT), jnp.float32),
               jax.ShapeDtypeStruct((_B, _OUT), jnp.float32)),
)


def kernel(input_ids, attention_mask, emb_g, W_g, b_g, emb_e, W_e, b_e,
           Wg1, bg1, g_gamma, g_beta, Wg2, bg2,
           Wc1, bc1, c_gamma, c_beta, Wc2, bc2):
    ids0 = input_ids[:, 0]
    m0 = attention_mask[:, 0].astype(jnp.float32).reshape(_B, 1)
    x_g, xe3 = _make_sc_gather()(emb_g, emb_e, ids0)

    out_c, out_p = _tc_dense(
        x_g, xe3, m0, W_g, b_g, W_e, b_e,
        Wg1, bg1, g_gamma, g_beta, Wg2, bg2,
        Wc1, bc1, c_gamma, c_beta, Wc2, bc2)
    return out_c, out_p


# speedup vs baseline: 1.0106x; 1.0106x over previous
import jax
import jax.numpy as jnp
from jax import lax
from jax.experimental import pallas as pl
from jax.experimental.pallas import tpu as pltpu, tpu_sc as plsc

_V = 30522
_D = 768
_H = 256
_B = 256
_E = 3
_OUT = 3
_PAD = 128
_NW = 32
_GG = _B // _NW
_GE = _E * _B // _NW


def _sc_gather_body(emb_g_hbm, emb_e_hbm, idx_hbm,
                    out_g_hbm, out_e_hbm,
                    idx_v, rowsg_v, rowse_v, semg, seme):
    wid = lax.axis_index("s") * 2 + lax.axis_index("c")
    pltpu.sync_copy(idx_hbm.at[pl.ds(wid * _GG, _GG)], idx_v)
    cp_g = pltpu.async_copy(emb_g_hbm.at[idx_v], rowsg_v, semg)
    cps = [pltpu.async_copy(emb_e_hbm.at[i].at[idx_v], rowse_v.at[i], seme)
           for i in range(_E)]
    cp_g.wait()
    pltpu.sync_copy(rowsg_v, out_g_hbm.at[pl.ds(wid * _GG, _GG)])
    for i in range(_E):
        cps[i].wait()
        pltpu.sync_copy(rowse_v.at[i], out_e_hbm.at[i, pl.ds(wid * _GG, _GG)])


def _make_sc_gather():
    return pl.kernel(
        _sc_gather_body,
        out_type=(jax.ShapeDtypeStruct((_B, _D), jnp.float32),
                  jax.ShapeDtypeStruct((_E, _B, _D), jnp.float32)),
        mesh=plsc.VectorSubcoreMesh(core_axis_name="c", subcore_axis_name="s"),
        scratch_types=[
            pltpu.VMEM((_GG,), jnp.int32),
            pltpu.VMEM((_GG, _D), jnp.float32),
            pltpu.VMEM((_E, _GG, _D), jnp.float32),
            pltpu.SemaphoreType.DMA,
            pltpu.SemaphoreType.DMA,
        ],
    )


def _ln_rows(x, gamma, beta):
    mu = jnp.mean(x, axis=-1, keepdims=True)
    v = jnp.mean((x - mu) ** 2, axis=-1, keepdims=True)
    return (x - mu) * lax.rsqrt(v + 1e-5) * gamma + beta


def _tc_body(xg_ref, xe_ref, m0_ref, Wg_ref, bg_ref, We_ref, be_ref,
             Wg1_ref, bg1_ref, gga_ref, gbe_ref, Wg2_ref, bg2_ref,
             Wc1_ref, bc1_ref, cga_ref, cbe_ref, Wc2_ref, bc2_ref,
             out_c_ref, out_p_ref):
    f32 = jnp.float32
    m0 = m0_ref[...]
    xg = xg_ref[...] * m0
    h = jax.nn.gelu(jnp.dot(xg, Wg_ref[...], preferred_element_type=f32)
                    + bg_ref[...].reshape(1, _D))
    g = (jnp.dot(h, Wg1_ref[...], preferred_element_type=f32)
         + bg1_ref[...].reshape(1, _H))
    g = jax.nn.gelu(_ln_rows(g, gga_ref[...].reshape(1, _H),
                             gbe_ref[...].reshape(1, _H)))
    logits = (jnp.dot(g, Wg2_ref[...], preferred_element_type=f32)
              + bg2_ref[...].reshape(1, _OUT))
    lmax = jnp.max(logits, axis=-1, keepdims=True)
    e = jnp.exp(logits - lmax)
    p = e / jnp.sum(e, axis=-1, keepdims=True)
    out_p_ref[...] = p
    p0, p1, p2 = p[:, 0:1], p[:, 1:2], p[:, 2:3]
    excl2 = (p2 <= p0) & (p2 <= p1)
    excl1 = jnp.logical_not(excl2) & (p1 <= p0) & (p1 < p2)
    excl0 = jnp.logical_not(excl2) & jnp.logical_not(excl1)
    w0 = jnp.where(excl0, 0.0, p0)
    w1 = jnp.where(excl1, 0.0, p1)
    w2 = jnp.where(excl2, 0.0, p2)
    denom = w0 + w1 + w2
    ws = (w0 / denom, w1 / denom, w2 / denom)
    acc = jnp.zeros((_B, _D), f32)
    for i in range(_E):
        xe = (xe_ref[i] * m0).astype(jnp.bfloat16)
        he = jax.nn.gelu(
            jnp.dot(xe, We_ref[i].astype(jnp.bfloat16),
                    preferred_element_type=f32)
            + be_ref[i].reshape(1, _D))
        acc = acc + he * ws[i]
    c = (jnp.dot(acc, Wc1_ref[...], preferred_element_type=f32)
         + bc1_ref[...].reshape(1, _H))
    c = jax.nn.gelu(_ln_rows(c, cga_ref[...].reshape(1, _H),
                             cbe_ref[...].reshape(1, _H)))
    out_c_ref[...] = (jnp.dot(c, Wc2_ref[...], preferred_element_type=f32)
                      + bc2_ref[...].reshape(1, _OUT))


_tc_dense = pl.pallas_call(
    _tc_body,
    out_shape=(jax.ShapeDtypeStruct((_B, _OUT), jnp.float32),
               jax.ShapeDtypeStruct((_B, _OUT), jnp.float32)),
)


def kernel(input_ids, attention_mask, emb_g, W_g, b_g, emb_e, W_e, b_e,
           Wg1, bg1, g_gamma, g_beta, Wg2, bg2,
           Wc1, bc1, c_gamma, c_beta, Wc2, bc2):
    ids0 = input_ids[:, 0]
    m0 = attention_mask[:, 0].astype(jnp.float32).reshape(_B, 1)
    x_g, xe3 = _make_sc_gather()(emb_g, emb_e, ids0)

    out_c, out_p = _tc_dense(
        x_g, xe3, m0, W_g, b_g, W_e, b_e,
        Wg1, bg1, g_gamma, g_beta, Wg2, bg2,
        Wc1, bc1, c_gamma, c_beta, Wc2, bc2)
    return out_c, out_p
